# baseline (speedup 1.0000x reference)
import functools

import jax
import jax.numpy as jnp
from jax import lax
from jax.experimental import pallas as pl
from jax.experimental.pallas import tpu as pltpu
from jax.experimental.pallas import tpu_sc as plsc

ALPHA = 0.1
BETA = 0.5
EPS = 1e-5

SUB = 64
BATCH = 32
NBATCH = 5
UNITS = BATCH * NBATCH
EDGES_PER_TILE = UNITS * SUB
E_PAD = 32 * EDGES_PER_TILE
ZROWS = 25


def _sc_scatter(feat, src2, dst2, mask2):
    n_nodes, d = feat.shape
    rows_per_tile = n_nodes // 16
    mesh = plsc.VectorSubcoreMesh(core_axis_name="c", subcore_axis_name="s")

    @functools.partial(
        pl.kernel,
        mesh=mesh,
        out_type=jax.ShapeDtypeStruct((2, n_nodes, d), jnp.float32),
        scratch_types=[
            pltpu.VMEM((BATCH, SUB), jnp.int32),
            pltpu.VMEM((BATCH, SUB), jnp.int32),
            pltpu.VMEM((BATCH, SUB), jnp.float32),
            pltpu.VMEM((2, SUB, d), jnp.float32),
            pltpu.VMEM((2, SUB, d), jnp.float32),
            pltpu.VMEM_SHARED((n_nodes, d), jnp.float32),
            pltpu.SemaphoreType.DMA,
            pltpu.SemaphoreType.DMA,
            pltpu.SemaphoreType.DMA,
            pltpu.SemaphoreType.DMA,
        ],
    )
    def k(feat_hbm, src_hbm, dst_hbm, mask_hbm, out_hbm,
          src_v, dst_v, mask_v, rin, rout, acc_sh,
          sem_g0, sem_g1, sem_s0, sem_s1):
        c = lax.axis_index("c")
        s = lax.axis_index("s")
        wid = s * 2 + c
        sem_g = (sem_g0, sem_g1)
        sem_s = (sem_s0, sem_s1)

        zeros16 = jnp.zeros((16,), jnp.float32)

        def zero_body(i, carry):
            rin[0, i // (d // 16), pl.ds((i % (d // 16)) * 16, 16)] = zeros16
            return carry

        lax.fori_loop(0, ZROWS * (d // 16), zero_body, 0)
        row0 = s * rows_per_tile
        for p in range(rows_per_tile // ZROWS):
            pltpu.sync_copy(rin.at[0, pl.ds(0, ZROWS)],
                            acc_sh.at[pl.ds(row0 + p * ZROWS, ZROWS)])
        plsc.subcore_barrier()

        def gather(u, p):
            return pltpu.async_copy(feat_hbm.at[src_v.at[u]], rin.at[p],
                                    sem_g[p])

        def scatter(u, p):
            return pltpu.async_copy(rout.at[p], acc_sh.at[dst_v.at[u]],
                                    sem_s[p], add=True)

        def drain_g(u, p):
            pltpu.make_async_copy(feat_hbm.at[src_v.at[u]], rin.at[p],
                                  sem_g[p]).wait()

        def drain_s(u, p):
            pltpu.make_async_copy(rout.at[p], acc_sh.at[dst_v.at[u]],
                                  sem_s[p]).wait()

        def scale(u, p):
            qsteps = d // 32

            def mul_body(gq, carry):
                gi = gq // qsteps
                q0 = (gq % qsteps) * 2
                mvec = mask_v[u, pl.ds(gi * 16, 16)]
                for t in range(16):
                    e = gi * 16 + t
                    mv = jnp.full((16,), mvec[t], dtype=jnp.float32)
                    for q in (q0, q0 + 1):
                        rout[p, e, pl.ds(q * 16, 16)] = (
                            rin[p, e, pl.ds(q * 16, 16)] * mv)
                return carry

            lax.fori_loop(0, (SUB // 16) * qsteps, mul_body, 0)

        unit0 = wid * UNITS

        def batch_body(b, carry):
            r = unit0 + b * BATCH
            pltpu.sync_copy(src_hbm.at[pl.ds(r, BATCH)], src_v)
            pltpu.sync_copy(dst_hbm.at[pl.ds(r, BATCH)], dst_v)
            pltpu.sync_copy(mask_hbm.at[pl.ds(r, BATCH)], mask_v)
            gather(0, 0)
            gather(1, 1)
            drain_g(0, 0)
            scale(0, 0)
            gather(2, 0)
            scatter(0, 0)
            drain_g(1, 1)
            scale(1, 1)
            gather(3, 1)
            scatter(1, 1)

            def pair_body(i, carry):
                for t in range(2):
                    u = 2 * i + t
                    drain_g(u, t)
                    drain_s(u - 2, t)
                    scale(u, t)
                    gather(u + 2, t)
                    scatter(u, t)
                return carry

            lax.fori_loop(1, BATCH // 2 - 1, pair_body, 0)

            for t in range(2):
                u = BATCH - 2 + t
                drain_g(u, t)
                drain_s(u - 2, t)
                scale(u, t)
                scatter(u, t)
            drain_s(BATCH - 2, 0)
            drain_s(BATCH - 1, 1)
            return carry

        lax.fori_loop(0, NBATCH, batch_body, 0)
        plsc.subcore_barrier()

        base = (n_nodes // (16 * 8)) * 8
        tail = n_nodes - 16 * base
        row0w = s * base
        pltpu.sync_copy(acc_sh.at[pl.ds(row0w, base)],
                        out_hbm.at[c, pl.ds(row0w, base)])
        if tail:
            @pl.when(s == 15)
            def _():
                pltpu.sync_copy(acc_sh.at[pl.ds(16 * base, tail)],
                                out_hbm.at[c, pl.ds(16 * base, tail)])

    return k(feat, src2, dst2, mask2)


def _tc_finish(h2, x0, W, gamma, beta):
    n_nodes, d = x0.shape

    def body(h2_ref, x0_ref, w_ref, g_ref, b_ref, o_ref):
        h = h2_ref[0] + h2_ref[1]
        mean = jnp.mean(h, axis=0, keepdims=True)
        dev = h - mean
        var = jnp.mean(dev * dev, axis=0, keepdims=True)
        hn = dev * lax.rsqrt(var + EPS) * g_ref[...] + b_ref[...]
        r = (1.0 - ALPHA) * hn + ALPHA * x0_ref[...]
        hw = lax.dot_general(r, w_ref[...], (((1,), (1,)), ((), ())),
                             preferred_element_type=jnp.float32,
                             precision=lax.Precision.HIGHEST)
        o_ref[...] = (1.0 - BETA) * r + BETA * hw

    return pl.pallas_call(
        body,
        out_shape=jax.ShapeDtypeStruct((n_nodes, d), jnp.float32),
    )(h2, x0, W, gamma.reshape(1, d), beta.reshape(1, d))


def kernel(features, initial_features, mask, W, bn_gamma, bn_beta, edge_index):
    e = edge_index.shape[1]
    pad = E_PAD - e
    src = jnp.pad(edge_index[0].astype(jnp.int32), (0, pad))
    dst = jnp.pad(edge_index[1].astype(jnp.int32), (0, pad))
    msk = jnp.pad(mask.astype(jnp.float32).reshape(e), (0, pad))
    src2 = src.reshape(E_PAD // SUB, SUB)
    dst2 = dst.reshape(E_PAD // SUB, SUB)
    mask2 = msk.reshape(E_PAD // SUB, SUB)
    h2 = _sc_scatter(features, src2, dst2, mask2)
    return _tc_finish(h2, initial_features, W, bn_gamma, bn_beta)


# speedup vs baseline: 1.1203x; 1.1203x over previous
import functools

import jax
import jax.numpy as jnp
from jax import lax
from jax.experimental import pallas as pl
from jax.experimental.pallas import tpu as pltpu
from jax.experimental.pallas import tpu_sc as plsc

ALPHA = 0.1
BETA = 0.5
EPS = 1e-5

SUB = 128
BATCH = 16
NBATCH = 5
UNITS = BATCH * NBATCH
EDGES_PER_TILE = UNITS * SUB
E_PAD = 32 * EDGES_PER_TILE
ZROWS = 125


def _sc_scatter(feat, src2, dst2, mask2):
    n_nodes, d = feat.shape
    rows_per_tile = n_nodes // 16
    mesh = plsc.VectorSubcoreMesh(core_axis_name="c", subcore_axis_name="s")

    @functools.partial(
        pl.kernel,
        mesh=mesh,
        out_type=jax.ShapeDtypeStruct((2, n_nodes, d), jnp.float32),
        scratch_types=[
            pltpu.VMEM((BATCH, SUB), jnp.int32),
            pltpu.VMEM((BATCH, SUB), jnp.int32),
            pltpu.VMEM((BATCH, SUB), jnp.float32),
            pltpu.VMEM((2, SUB, d), jnp.float32),
            pltpu.VMEM_SHARED((n_nodes, d), jnp.float32),
            pltpu.SemaphoreType.DMA,
            pltpu.SemaphoreType.DMA,
        ],
    )
    def k(feat_hbm, src_hbm, dst_hbm, mask_hbm, out_hbm,
          src_v, dst_v, mask_v, rin, acc_sh, sem_g0, sem_g1):
        c = lax.axis_index("c")
        s = lax.axis_index("s")
        wid = s * 2 + c
        sem_g = (sem_g0, sem_g1)

        zeros16 = jnp.zeros((16,), jnp.float32)

        def zero_body(i, carry):
            rin[0, i // (d // 16), pl.ds((i % (d // 16)) * 16, 16)] = zeros16
            return carry

        lax.fori_loop(0, ZROWS * (d // 16), zero_body, 0)
        row0 = s * rows_per_tile
        for p in range(rows_per_tile // ZROWS):
            pltpu.sync_copy(rin.at[0, pl.ds(0, ZROWS)],
                            acc_sh.at[pl.ds(row0 + p * ZROWS, ZROWS)])
        plsc.subcore_barrier()

        def gather(u, p):
            pltpu.async_copy(feat_hbm.at[src_v.at[u]], rin.at[p], sem_g[p])

        def drain_g(u, p):
            pltpu.make_async_copy(feat_hbm.at[src_v.at[u]], rin.at[p],
                                  sem_g[p]).wait()

        def scatter(u, p):
            pltpu.sync_copy(rin.at[p], acc_sh.at[dst_v.at[u]], add=True)

        def scale(u, p):
            @plsc.parallel_loop(0, SUB // 16)
            def mul_body(gi):
                mvec = mask_v[u, pl.ds(gi * 16, 16)]
                for t in range(16):
                    e = gi * 16 + t
                    mv = jnp.full((16,), mvec[t], dtype=jnp.float32)
                    for q in range(d // 16):
                        rin[p, e, pl.ds(q * 16, 16)] = (
                            rin[p, e, pl.ds(q * 16, 16)] * mv)

        unit0 = wid * UNITS

        def batch_body(b, carry):
            r = unit0 + b * BATCH
            pltpu.sync_copy(src_hbm.at[pl.ds(r, BATCH)], src_v)
            pltpu.sync_copy(dst_hbm.at[pl.ds(r, BATCH)], dst_v)
            pltpu.sync_copy(mask_hbm.at[pl.ds(r, BATCH)], mask_v)
            gather(0, 0)
            gather(1, 1)
            for t in range(2):
                drain_g(t, t)
                scale(t, t)
                scatter(t, t)
                gather(t + 2, t)

            def pair_body(i, carry):
                for t in range(2):
                    u = 2 * i + t
                    drain_g(u, t)
                    scale(u, t)
                    scatter(u, t)
                    gather(u + 2, t)
                return carry

            lax.fori_loop(1, BATCH // 2 - 1, pair_body, 0)

            for t in range(2):
                u = BATCH - 2 + t
                drain_g(u, t)
                scale(u, t)
                scatter(u, t)
            return carry

        lax.fori_loop(0, NBATCH, batch_body, 0)
        plsc.subcore_barrier()

        base = (n_nodes // (16 * 8)) * 8
        tail = n_nodes - 16 * base
        row0w = s * base
        pltpu.sync_copy(acc_sh.at[pl.ds(row0w, base)],
                        out_hbm.at[c, pl.ds(row0w, base)])
        if tail:
            @pl.when(s == 15)
            def _():
                pltpu.sync_copy(acc_sh.at[pl.ds(16 * base, tail)],
                                out_hbm.at[c, pl.ds(16 * base, tail)])

    return k(feat, src2, dst2, mask2)


def _tc_finish(h2, x0, W, gamma, beta):
    n_nodes, d = x0.shape

    def body(h2_ref, x0_ref, w_ref, g_ref, b_ref, o_ref):
        h = h2_ref[0] + h2_ref[1]
        mean = jnp.mean(h, axis=0, keepdims=True)
        dev = h - mean
        var = jnp.mean(dev * dev, axis=0, keepdims=True)
        hn = dev * lax.rsqrt(var + EPS) * g_ref[...] + b_ref[...]
        r = (1.0 - ALPHA) * hn + ALPHA * x0_ref[...]
        hw = lax.dot_general(r, w_ref[...], (((1,), (1,)), ((), ())),
                             preferred_element_type=jnp.float32,
                             precision=lax.Precision.HIGHEST)
        o_ref[...] = (1.0 - BETA) * r + BETA * hw

    return pl.pallas_call(
        body,
        out_shape=jax.ShapeDtypeStruct((n_nodes, d), jnp.float32),
    )(h2, x0, W, gamma.reshape(1, d), beta.reshape(1, d))


def kernel(features, initial_features, mask, W, bn_gamma, bn_beta, edge_index):
    e = edge_index.shape[1]
    pad = E_PAD - e
    src = jnp.pad(edge_index[0].astype(jnp.int32), (0, pad))
    dst = jnp.pad(edge_index[1].astype(jnp.int32), (0, pad))
    msk = jnp.pad(mask.astype(jnp.float32).reshape(e), (0, pad))
    src2 = src.reshape(E_PAD // SUB, SUB)
    dst2 = dst.reshape(E_PAD // SUB, SUB)
    mask2 = msk.reshape(E_PAD // SUB, SUB)
    h2 = _sc_scatter(features, src2, dst2, mask2)
    return _tc_finish(h2, initial_features, W, bn_gamma, bn_beta)


# speedup vs baseline: 1.8573x; 1.6578x over previous
import functools

import jax
import jax.numpy as jnp
from jax import lax
from jax.experimental import pallas as pl
from jax.experimental.pallas import tpu as pltpu
from jax.experimental.pallas import tpu_sc as plsc

ALPHA = 0.1
BETA = 0.5
EPS = 1e-5

SUB = 128
NSUB = 2
CHUNK = SUB * NSUB
NW = 32
ZROWS = 125


def _sc_scatter(features, src2, dst2, mask2):
    n_nodes, d = features.shape
    num_chunks = (src2.shape[0] * src2.shape[1]) // CHUNK
    rows_per_tile = n_nodes // 16
    mesh = plsc.VectorSubcoreMesh(core_axis_name="c", subcore_axis_name="s")

    @functools.partial(
        pl.kernel,
        mesh=mesh,
        out_type=jax.ShapeDtypeStruct((2, n_nodes, d), jnp.float32),
        scratch_types=[
            pltpu.VMEM((NSUB, SUB), jnp.int32),
            pltpu.VMEM((NSUB, SUB), jnp.int32),
            pltpu.VMEM((NSUB, SUB), jnp.float32),
            pltpu.VMEM((CHUNK, d), jnp.float32),
            pltpu.VMEM_SHARED((n_nodes, d), jnp.float32),
            pltpu.SemaphoreType.DMA,
        ],
    )
    def k(feat_hbm, src_hbm, dst_hbm, mask_hbm, out_hbm,
          src_v, dst_v, mask_v, rows_v, acc_sh, sem):
        c = lax.axis_index("c")
        s = lax.axis_index("s")
        wid = s * 2 + c

        zeros16 = jnp.zeros((16,), jnp.float32)

        def zero_body(i, carry):
            rows_v[i // 8, pl.ds((i % 8) * 16, 16)] = zeros16
            return carry

        lax.fori_loop(0, ZROWS * (d // 16), zero_body, 0)
        row0 = s * rows_per_tile
        for p in range(rows_per_tile // ZROWS):
            pltpu.sync_copy(rows_v.at[pl.ds(0, ZROWS)],
                            acc_sh.at[pl.ds(row0 + p * ZROWS, ZROWS)])
        plsc.subcore_barrier()

        n_my = (num_chunks - wid + NW - 1) // NW

        def chunk_body(t, carry):
            g = wid + t * NW
            pltpu.sync_copy(src_hbm.at[pl.ds(g * NSUB, NSUB)], src_v)
            pltpu.sync_copy(dst_hbm.at[pl.ds(g * NSUB, NSUB)], dst_v)
            pltpu.sync_copy(mask_hbm.at[pl.ds(g * NSUB, NSUB)], mask_v)
            for j in range(NSUB):
                rows_j = rows_v.at[pl.ds(j * SUB, SUB)]
                pltpu.async_copy(feat_hbm.at[src_v.at[j]], rows_j, sem).wait()

                @plsc.parallel_loop(0, SUB // 16)
                def mul_body(gi, j=j):
                    mvec = mask_v[j, pl.ds(gi * 16, 16)]
                    for t in range(16):
                        e = j * SUB + gi * 16 + t
                        mv = jnp.full((16,), mvec[t], dtype=jnp.float32)
                        for q in range(d // 16):
                            rows_v[e, pl.ds(q * 16, 16)] = (
                                rows_v[e, pl.ds(q * 16, 16)] * mv)
                pltpu.sync_copy(rows_j, acc_sh.at[dst_v.at[j]], add=True)
            return carry

        lax.fori_loop(0, n_my, chunk_body, 0)
        plsc.subcore_barrier()

        base = (n_nodes // (16 * 8)) * 8
        tail = n_nodes - 16 * base
        row0w = s * base
        pltpu.sync_copy(acc_sh.at[pl.ds(row0w, base)],
                        out_hbm.at[c, pl.ds(row0w, base)])
        if tail:
            @pl.when(s == 15)
            def _():
                pltpu.sync_copy(acc_sh.at[pl.ds(16 * base, tail)],
                                out_hbm.at[c, pl.ds(16 * base, tail)])

    return k(features, src2, dst2, mask2)


def _tc_finish(h2, x0, W, gamma, beta):
    n_nodes, d = x0.shape

    def body(h2_ref, x0_ref, w_ref, g_ref, b_ref, o_ref):
        h = h2_ref[0] + h2_ref[1]
        mean = jnp.mean(h, axis=0, keepdims=True)
        dev = h - mean
        var = jnp.mean(dev * dev, axis=0, keepdims=True)
        hn = dev * lax.rsqrt(var + EPS) * g_ref[...] + b_ref[...]
        r = (1.0 - ALPHA) * hn + ALPHA * x0_ref[...]
        hw = lax.dot_general(r, w_ref[...], (((1,), (1,)), ((), ())),
                             preferred_element_type=jnp.float32,
                             precision=lax.Precision.HIGHEST)
        o_ref[...] = (1.0 - BETA) * r + BETA * hw

    return pl.pallas_call(
        body,
        out_shape=jax.ShapeDtypeStruct((n_nodes, d), jnp.float32),
    )(h2, x0, W, gamma.reshape(1, d), beta.reshape(1, d))


def kernel(features, initial_features, mask, W, bn_gamma, bn_beta, edge_index):
    e = edge_index.shape[1]
    src2 = edge_index[0].astype(jnp.int32).reshape(e // SUB, SUB)
    dst2 = edge_index[1].astype(jnp.int32).reshape(e // SUB, SUB)
    mask2 = mask.astype(jnp.float32).reshape(e // SUB, SUB)
    h2 = _sc_scatter(features, src2, dst2, mask2)
    return _tc_finish(h2, initial_features, W, bn_gamma, bn_beta)
